# initial kernel scaffold (unmeasured)
import jax
import jax.numpy as jnp
from jax import lax
from jax.experimental import pallas as pl
from jax.experimental.pallas import tpu as pltpu

N_DEV = 8
KW = 4
HALO = KW - 1
CHUNK = 512


def kernel(x, k):
    b, s, c = x.shape
    n_chunks = s // CHUNK

    def body(x_ref, k_ref, out_ref, halo_ref, send_ref, send_sem, recv_sem):
        my = lax.axis_index("i")
        left = my - 1
        right = my + 1

        send_ref[...] = x_ref[:, s - HALO:, :]

        barrier_sem = pltpu.get_barrier_semaphore()

        @pl.when(my > 0)
        def _signal_left():
            pl.semaphore_signal(
                barrier_sem, inc=1,
                device_id=(left,), device_id_type=pl.DeviceIdType.MESH,
            )

        @pl.when(my < N_DEV - 1)
        def _send_right():
            pl.semaphore_wait(barrier_sem, 1)
            rdma = pltpu.make_async_remote_copy(
                src_ref=send_ref, dst_ref=halo_ref,
                send_sem=send_sem, recv_sem=recv_sem,
                device_id=(right,), device_id_type=pl.DeviceIdType.MESH,
            )
            rdma.start()
            rdma.wait_send()

        @pl.when(my == 0)
        def _zero_halo():
            halo_ref[...] = jnp.zeros_like(halo_ref)

        @pl.when(my > 0)
        def _recv_left():
            recv = pltpu.make_async_remote_copy(
                src_ref=send_ref, dst_ref=halo_ref,
                send_sem=send_sem, recv_sem=recv_sem,
                device_id=(left,), device_id_type=pl.DeviceIdType.MESH,
            )
            recv.wait_recv()

        kv = k_ref[...]
        taps = [kv[t:t + 1, :].reshape(1, 1, c) for t in range(KW)]

        def conv_silu(seg):
            acc = seg[:, HALO:, :] * taps[KW - 1]
            for t in range(KW - 1):
                acc = acc + seg[:, t:t + CHUNK, :] * taps[t]
            return acc * jax.nn.sigmoid(acc)

        seg0 = jnp.concatenate([halo_ref[...], x_ref[:, :CHUNK, :]], axis=1)
        out_ref[:, :CHUNK, :] = conv_silu(seg0)

        def chunk_body(ci, carry):
            start = ci * CHUNK
            seg = x_ref[:, pl.ds(start - HALO, CHUNK + HALO), :]
            out_ref[:, pl.ds(start, CHUNK), :] = conv_silu(seg)
            return carry

        lax.fori_loop(1, n_chunks, chunk_body, 0)

    return pl.pallas_call(
        body,
        out_shape=jax.ShapeDtypeStruct((b, s, c), x.dtype),
        in_specs=[
            pl.BlockSpec(memory_space=pltpu.VMEM),
            pl.BlockSpec(memory_space=pltpu.VMEM),
        ],
        out_specs=pl.BlockSpec(memory_space=pltpu.VMEM),
        scratch_shapes=[
            pltpu.VMEM((b, HALO, c), x.dtype),
            pltpu.VMEM((b, HALO, c), x.dtype),
            pltpu.SemaphoreType.DMA,
            pltpu.SemaphoreType.DMA,
        ],
        compiler_params=pltpu.CompilerParams(collective_id=0),
    )(x, k)


# baseline (device time: 59178 ns/iter reference)
import jax
import jax.numpy as jnp
from jax import lax
from jax.experimental import pallas as pl
from jax.experimental.pallas import tpu as pltpu

N_DEV = 8
KW = 4
HALO = KW - 1
CHUNK = 512


def kernel(x, k):
    b, s, c = x.shape
    n_chunks = s // CHUNK

    def body(x_ref, k_ref, out_ref, halo_ref, send_ref, send_sem, recv_sem):
        my = lax.axis_index("i")
        left = my - 1
        right = my + 1

        send_ref[...] = x_ref[:, s - HALO:, :]

        barrier_sem = pltpu.get_barrier_semaphore()

        @pl.when(my > 0)
        def _signal_left():
            pl.semaphore_signal(
                barrier_sem, inc=1,
                device_id=(left,), device_id_type=pl.DeviceIdType.MESH,
            )

        @pl.when(my < N_DEV - 1)
        def _send_right():
            pl.semaphore_wait(barrier_sem, 1)
            rdma = pltpu.make_async_remote_copy(
                src_ref=send_ref, dst_ref=halo_ref,
                send_sem=send_sem, recv_sem=recv_sem,
                device_id=(right,), device_id_type=pl.DeviceIdType.MESH,
            )
            rdma.start()
            rdma.wait_send()

        @pl.when(my == 0)
        def _zero_halo():
            halo_ref[...] = jnp.zeros_like(halo_ref)

        @pl.when(my > 0)
        def _recv_left():
            recv = pltpu.make_async_remote_copy(
                src_ref=send_ref, dst_ref=halo_ref,
                send_sem=send_sem, recv_sem=recv_sem,
                device_id=(left,), device_id_type=pl.DeviceIdType.MESH,
            )
            recv.wait_recv()

        kv = k_ref[...]
        taps = [kv[t:t + 1, :].reshape(1, 1, c) for t in range(KW)]

        def conv_silu(seg):
            acc = seg[:, HALO:, :] * taps[KW - 1]
            for t in range(KW - 1):
                acc = acc + seg[:, t:t + CHUNK, :] * taps[t]
            return (acc * jax.nn.sigmoid(acc)).astype(jnp.bfloat16)

        seg0 = jnp.concatenate([halo_ref[...], x_ref[:, :CHUNK, :]], axis=1)
        out_ref[:, :CHUNK, :] = conv_silu(seg0)

        def chunk_body(ci, carry):
            start = ci * CHUNK
            w = x_ref[:, pl.ds(start - 8, CHUNK + 8), :]
            seg = w[:, 8 - HALO:, :]
            out_ref[:, pl.ds(start, CHUNK), :] = conv_silu(seg)
            return carry

        lax.fori_loop(1, n_chunks, chunk_body, 0)

    return pl.pallas_call(
        body,
        out_shape=jax.ShapeDtypeStruct((b, s, c), jnp.bfloat16),
        in_specs=[
            pl.BlockSpec(memory_space=pltpu.VMEM),
            pl.BlockSpec(memory_space=pltpu.VMEM),
        ],
        out_specs=pl.BlockSpec(memory_space=pltpu.VMEM),
        scratch_shapes=[
            pltpu.VMEM((b, HALO, c), x.dtype),
            pltpu.VMEM((b, HALO, c), x.dtype),
            pltpu.SemaphoreType.DMA,
            pltpu.SemaphoreType.DMA,
        ],
        compiler_params=pltpu.CompilerParams(
            collective_id=0,
            vmem_limit_bytes=60 * 1024 * 1024,
        ),
    )(x, k)


# device time: 46538 ns/iter; 1.2716x vs baseline; 1.2716x over previous
import jax
import jax.numpy as jnp
from jax import lax
from jax.experimental import pallas as pl
from jax.experimental.pallas import tpu as pltpu

N_DEV = 8
KW = 4
HALO = KW - 1
PAD = 8
CHUNK = 512


def kernel(x, k):
    b, s, c = x.shape
    n_chunks = s // CHUNK

    def body(x_hbm, k_ref, out_hbm, xbuf, obuf, send_ref,
             in_sems, out_sems, send_sem, recv_sem, stage_sem):
        my = lax.axis_index("i")
        left = my - 1
        right = my + 1

        def in_copy(ci, slot):
            if ci == 0:
                return pltpu.make_async_copy(
                    x_hbm.at[:, pl.ds(0, CHUNK), :],
                    xbuf.at[slot, :, pl.ds(PAD, CHUNK), :],
                    in_sems.at[slot],
                )
            return pltpu.make_async_copy(
                x_hbm.at[:, pl.ds(ci * CHUNK - PAD, CHUNK + PAD), :],
                xbuf.at[slot],
                in_sems.at[slot],
            )

        def out_copy(ci, slot):
            return pltpu.make_async_copy(
                obuf.at[slot],
                out_hbm.at[:, pl.ds(ci * CHUNK, CHUNK), :],
                out_sems.at[slot],
            )

        in_copy(0, 0).start()
        in_copy(1, 1).start()
        stage = pltpu.make_async_copy(
            x_hbm.at[:, pl.ds(s - PAD, PAD), :], send_ref, stage_sem,
        )
        stage.start()

        barrier_sem = pltpu.get_barrier_semaphore()

        @pl.when(my > 0)
        def _signal_left():
            pl.semaphore_signal(
                barrier_sem, inc=1,
                device_id=(left,), device_id_type=pl.DeviceIdType.MESH,
            )

        @pl.when(my < N_DEV - 1)
        def _send_right():
            pl.semaphore_wait(barrier_sem, 1)
            stage.wait()
            rdma = pltpu.make_async_remote_copy(
                src_ref=send_ref,
                dst_ref=xbuf.at[0, :, pl.ds(0, PAD), :],
                send_sem=send_sem, recv_sem=recv_sem,
                device_id=(right,), device_id_type=pl.DeviceIdType.MESH,
            )
            rdma.start()
            rdma.wait_send()

        @pl.when(my == N_DEV - 1)
        def _drain_stage():
            stage.wait()

        @pl.when(my == 0)
        def _zero_halo():
            xbuf[0, :, PAD - HALO:PAD, :] = jnp.zeros(
                (b, HALO, c), xbuf.dtype
            )

        @pl.when(my > 0)
        def _recv_left():
            recv = pltpu.make_async_remote_copy(
                src_ref=send_ref,
                dst_ref=xbuf.at[0, :, pl.ds(0, PAD), :],
                send_sem=send_sem, recv_sem=recv_sem,
                device_id=(left,), device_id_type=pl.DeviceIdType.MESH,
            )
            recv.wait_recv()

        kv = k_ref[...]
        taps = [kv[t:t + 1, :].reshape(1, 1, c) for t in range(KW)]

        def conv_silu(seg):
            acc = seg[:, HALO:, :] * taps[KW - 1]
            for t in range(KW - 1):
                acc = acc + seg[:, t:t + CHUNK, :] * taps[t]
            return (acc * jax.nn.sigmoid(acc)).astype(jnp.bfloat16)

        for ci in range(n_chunks):
            slot = ci % 2
            if ci + 1 < n_chunks and ci >= 1:
                in_copy(ci + 1, (ci + 1) % 2).start()
            in_copy(ci, slot).wait()
            if ci >= 2:
                out_copy(ci - 2, slot).wait()
            seg = xbuf[slot, :, PAD - HALO:, :]
            obuf[slot] = conv_silu(seg)
            out_copy(ci, slot).start()
        out_copy(n_chunks - 2, (n_chunks - 2) % 2).wait()
        out_copy(n_chunks - 1, (n_chunks - 1) % 2).wait()

    return pl.pallas_call(
        body,
        out_shape=jax.ShapeDtypeStruct((b, s, c), jnp.bfloat16),
        in_specs=[
            pl.BlockSpec(memory_space=pl.ANY),
            pl.BlockSpec(memory_space=pltpu.VMEM),
        ],
        out_specs=pl.BlockSpec(memory_space=pl.ANY),
        scratch_shapes=[
            pltpu.VMEM((2, b, CHUNK + PAD, c), x.dtype),
            pltpu.VMEM((2, b, CHUNK, c), jnp.bfloat16),
            pltpu.VMEM((b, PAD, c), x.dtype),
            pltpu.SemaphoreType.DMA((2,)),
            pltpu.SemaphoreType.DMA((2,)),
            pltpu.SemaphoreType.DMA,
            pltpu.SemaphoreType.DMA,
            pltpu.SemaphoreType.DMA,
        ],
        compiler_params=pltpu.CompilerParams(
            collective_id=0,
            vmem_limit_bytes=60 * 1024 * 1024,
        ),
    )(x, k)


# device time: 41785 ns/iter; 1.4162x vs baseline; 1.1137x over previous
import jax
import jax.numpy as jnp
from jax import lax
from jax.experimental import pallas as pl
from jax.experimental.pallas import tpu as pltpu

N_DEV = 8
KW = 4
HALO = KW - 1
PAD = 8
CHUNK = 512


def kernel(x, k):
    b, s, c = x.shape
    n_chunks = s // CHUNK

    def body(x_hbm, k_ref, out_hbm, xbuf, obuf, send_ref,
             in_sems, out_sems, send_sem, recv_sem, stage_sem):
        my = lax.axis_index("i")
        left = my - 1
        right = my + 1

        def in_copy(ci, slot):
            if ci == 0:
                return pltpu.make_async_copy(
                    x_hbm.at[:, pl.ds(0, CHUNK), :],
                    xbuf.at[slot, :, pl.ds(PAD, CHUNK), :],
                    in_sems.at[slot],
                )
            return pltpu.make_async_copy(
                x_hbm.at[:, pl.ds(ci * CHUNK - PAD, CHUNK + PAD), :],
                xbuf.at[slot],
                in_sems.at[slot],
            )

        def out_copy(ci, slot):
            return pltpu.make_async_copy(
                obuf.at[slot],
                out_hbm.at[:, pl.ds(ci * CHUNK, CHUNK), :],
                out_sems.at[slot],
            )

        in_copy(0, 0).start()
        in_copy(1, 1).start()
        stage = pltpu.make_async_copy(
            x_hbm.at[:, pl.ds(s - PAD, PAD), :], send_ref, stage_sem,
        )
        stage.start()

        barrier_sem = pltpu.get_barrier_semaphore()

        @pl.when(my > 0)
        def _signal_left():
            pl.semaphore_signal(
                barrier_sem, inc=1,
                device_id=(left,), device_id_type=pl.DeviceIdType.MESH,
            )

        @pl.when(my < N_DEV - 1)
        def _send_right():
            pl.semaphore_wait(barrier_sem, 1)
            stage.wait()
            rdma = pltpu.make_async_remote_copy(
                src_ref=send_ref,
                dst_ref=xbuf.at[0, :, pl.ds(0, PAD), :],
                send_sem=send_sem, recv_sem=recv_sem,
                device_id=(right,), device_id_type=pl.DeviceIdType.MESH,
            )
            rdma.start()
            rdma.wait_send()

        @pl.when(my == N_DEV - 1)
        def _drain_stage():
            stage.wait()

        @pl.when(my == 0)
        def _zero_halo():
            xbuf[0, :, PAD - HALO:PAD, :] = jnp.zeros(
                (b, HALO, c), xbuf.dtype
            )

        @pl.when(my > 0)
        def _recv_left():
            recv = pltpu.make_async_remote_copy(
                src_ref=send_ref,
                dst_ref=xbuf.at[0, :, pl.ds(0, PAD), :],
                send_sem=send_sem, recv_sem=recv_sem,
                device_id=(left,), device_id_type=pl.DeviceIdType.MESH,
            )
            recv.wait_recv()

        kv = k_ref[...].astype(jnp.bfloat16)
        taps = [kv[t:t + 1, :].reshape(1, 1, c) for t in range(KW)]

        def conv_silu(seg):
            seg = seg.astype(jnp.bfloat16)
            acc = seg[:, HALO:, :] * taps[KW - 1]
            for t in range(KW - 1):
                acc = acc + seg[:, t:t + CHUNK, :] * taps[t]
            return acc * jax.nn.sigmoid(acc)

        for ci in range(n_chunks):
            slot = ci % 2
            if ci + 1 < n_chunks and ci >= 1:
                in_copy(ci + 1, (ci + 1) % 2).start()
            in_copy(ci, slot).wait()
            if ci >= 2:
                out_copy(ci - 2, slot).wait()
            seg = xbuf[slot, :, PAD - HALO:, :]
            obuf[slot] = conv_silu(seg)
            out_copy(ci, slot).start()
        out_copy(n_chunks - 2, (n_chunks - 2) % 2).wait()
        out_copy(n_chunks - 1, (n_chunks - 1) % 2).wait()

    return pl.pallas_call(
        body,
        out_shape=jax.ShapeDtypeStruct((b, s, c), jnp.bfloat16),
        in_specs=[
            pl.BlockSpec(memory_space=pl.ANY),
            pl.BlockSpec(memory_space=pltpu.VMEM),
        ],
        out_specs=pl.BlockSpec(memory_space=pl.ANY),
        scratch_shapes=[
            pltpu.VMEM((2, b, CHUNK + PAD, c), x.dtype),
            pltpu.VMEM((2, b, CHUNK, c), jnp.bfloat16),
            pltpu.VMEM((b, PAD, c), x.dtype),
            pltpu.SemaphoreType.DMA((2,)),
            pltpu.SemaphoreType.DMA((2,)),
            pltpu.SemaphoreType.DMA,
            pltpu.SemaphoreType.DMA,
            pltpu.SemaphoreType.DMA,
        ],
        compiler_params=pltpu.CompilerParams(
            collective_id=0,
            vmem_limit_bytes=60 * 1024 * 1024,
        ),
    )(x, k)


# device time: 38110 ns/iter; 1.5528x vs baseline; 1.0964x over previous
import jax
import jax.numpy as jnp
from jax import lax
from jax.experimental import pallas as pl
from jax.experimental.pallas import tpu as pltpu

N_DEV = 8
KW = 4
HALO = KW - 1
PAD = 8
CHUNK = 256
NBUF = 8


def kernel(x, k):
    b, s, c = x.shape
    n_chunks = s // CHUNK
    order = list(range(1, n_chunks)) + [0]

    def body(x_hbm, k_ref, out_hbm, xbuf, obuf, halo_ref, send_ref,
             in_sems, out_sems, send_sem, recv_sem, stage_sem):
        my = lax.axis_index("i")
        left = my - 1
        right = my + 1

        def in_copy(ci, slot):
            if ci == 0:
                return pltpu.make_async_copy(
                    x_hbm.at[:, pl.ds(0, CHUNK), :],
                    xbuf.at[slot, :, pl.ds(PAD, CHUNK), :],
                    in_sems.at[slot],
                )
            return pltpu.make_async_copy(
                x_hbm.at[:, pl.ds(ci * CHUNK - PAD, CHUNK + PAD), :],
                xbuf.at[slot],
                in_sems.at[slot],
            )

        def out_copy(ci, slot):
            return pltpu.make_async_copy(
                obuf.at[slot],
                out_hbm.at[:, pl.ds(ci * CHUNK, CHUNK), :],
                out_sems.at[slot],
            )

        def halo_rdma(peer):
            return pltpu.make_async_remote_copy(
                src_ref=send_ref, dst_ref=halo_ref,
                send_sem=send_sem, recv_sem=recv_sem,
                device_id=(peer,), device_id_type=pl.DeviceIdType.MESH,
            )

        for pos in range(min(NBUF, n_chunks)):
            in_copy(order[pos], pos % NBUF).start()
        stage = pltpu.make_async_copy(
            x_hbm.at[:, pl.ds(s - PAD, PAD), :], send_ref, stage_sem,
        )
        stage.start()

        barrier_sem = pltpu.get_barrier_semaphore()

        @pl.when(my > 0)
        def _signal_left():
            pl.semaphore_signal(
                barrier_sem, inc=1,
                device_id=(left,), device_id_type=pl.DeviceIdType.MESH,
            )

        @pl.when(my < N_DEV - 1)
        def _send_right():
            pl.semaphore_wait(barrier_sem, 1)
            stage.wait()
            halo_rdma(right).start()

        @pl.when(my == N_DEV - 1)
        def _drain_stage():
            stage.wait()

        @pl.when(my == 0)
        def _zero_halo():
            halo_ref[...] = jnp.zeros((b, PAD, c), halo_ref.dtype)

        kv = k_ref[...].astype(jnp.bfloat16)
        taps = [kv[t:t + 1, :].reshape(1, 1, c) for t in range(KW)]

        def conv_silu(seg):
            seg = seg.astype(jnp.bfloat16)
            acc = seg[:, HALO:, :] * taps[KW - 1]
            for t in range(KW - 1):
                acc = acc + seg[:, t:t + CHUNK, :] * taps[t]
            return acc * (0.5 * jnp.tanh(0.5 * acc) + 0.5)

        for pos in range(n_chunks):
            ci = order[pos]
            slot = pos % NBUF
            in_copy(ci, slot).wait()
            if ci == 0:
                @pl.when(my > 0)
                def _recv_left():
                    halo_rdma(left).wait_recv()

                xbuf[slot, :, PAD - HALO:PAD, :] = halo_ref[:, PAD - HALO:, :]
            if pos >= NBUF:
                out_copy(order[pos - NBUF], slot).wait()
            seg = xbuf[slot, :, PAD - HALO:, :]
            obuf[slot] = conv_silu(seg)
            out_copy(ci, slot).start()
            if pos + NBUF < n_chunks:
                in_copy(order[pos + NBUF], slot).start()
        for pos in range(max(0, n_chunks - NBUF), n_chunks):
            out_copy(order[pos], pos % NBUF).wait()

        @pl.when(my < N_DEV - 1)
        def _drain_send():
            halo_rdma(right).wait_send()

    return pl.pallas_call(
        body,
        out_shape=jax.ShapeDtypeStruct((b, s, c), jnp.bfloat16),
        in_specs=[
            pl.BlockSpec(memory_space=pl.ANY),
            pl.BlockSpec(memory_space=pltpu.VMEM),
        ],
        out_specs=pl.BlockSpec(memory_space=pl.ANY),
        scratch_shapes=[
            pltpu.VMEM((NBUF, b, CHUNK + PAD, c), x.dtype),
            pltpu.VMEM((NBUF, b, CHUNK, c), jnp.bfloat16),
            pltpu.VMEM((b, PAD, c), x.dtype),
            pltpu.VMEM((b, PAD, c), x.dtype),
            pltpu.SemaphoreType.DMA((NBUF,)),
            pltpu.SemaphoreType.DMA((NBUF,)),
            pltpu.SemaphoreType.DMA,
            pltpu.SemaphoreType.DMA,
            pltpu.SemaphoreType.DMA,
        ],
        compiler_params=pltpu.CompilerParams(
            collective_id=0,
            vmem_limit_bytes=60 * 1024 * 1024,
        ),
    )(x, k)
